# baseline (speedup 1.0000x reference)
import jax
import jax.numpy as jnp
from jax import lax
from jax.experimental import pallas as pl
from jax.experimental.pallas import tpu as pltpu
from jax.experimental.pallas import tpu_sc as plsc

N = 10000
E = 320000
D = 128
H = 128
C = 64

NC = 2
NS = 16
NW = NC * NS

N_PAD = 10240
E_PAD = 327680
CHUNK = 128
ROWS_PER_TILE = N_PAD // NS

EPT0 = 18432
EPT1 = (E_PAD - NS * EPT0) // NS
STAGES0 = 3
STAGES1 = 1
IDX_ROWS0 = EPT0 // STAGES0 // CHUNK
IDX_ROWS1 = 16
N_GROUPS0 = IDX_ROWS0 // 2
N_GROUPS1 = IDX_ROWS1 // 2
CORE1_ROW0 = NS * EPT0 // CHUNK


def _segsum_body(src_hbm, dst_hbm, table_hbm, zeros_hbm, out_hbm,
                 src_idx, dst_idx, rows0, rows1, acc, sem0, sem1):
    cid = lax.axis_index("c")
    sid = lax.axis_index("s")

    pltpu.sync_copy(zeros_hbm, acc.at[pl.ds(sid * ROWS_PER_TILE, ROWS_PER_TILE)])
    plsc.subcore_barrier()

    def fire(c, slot, sem):
        pltpu.async_copy(table_hbm.at[src_idx.at[c]], slot, sem)

    def drain(c, slot, sem):
        pltpu.make_async_copy(table_hbm.at[src_idx.at[c]], slot, sem).wait()

    def scatter(c, slot):
        pltpu.sync_copy(slot, acc.at[dst_idx.at[c]], add=True)

    def run(base_row, n_stages, idx_rows, n_groups):
        for stage in range(n_stages):
            idx_row0 = base_row + stage * idx_rows
            pltpu.sync_copy(src_hbm.at[pl.ds(idx_row0, idx_rows)],
                            src_idx.at[pl.ds(0, idx_rows)])
            pltpu.sync_copy(dst_hbm.at[pl.ds(idx_row0, idx_rows)],
                            dst_idx.at[pl.ds(0, idx_rows)])

            fire(0, rows0, sem0)

            def group_body(g, carry):
                c0 = 2 * g
                fire(c0 + 1, rows1, sem1)
                drain(c0, rows0, sem0)
                scatter(c0, rows0)

                @pl.when(g < n_groups - 1)
                def _():
                    fire(c0 + 2, rows0, sem0)

                drain(c0 + 1, rows1, sem1)
                scatter(c0 + 1, rows1)
                return carry

            lax.fori_loop(0, n_groups, group_body, 0)

    @pl.when(cid == 0)
    def _():
        run(sid * (EPT0 // CHUNK), STAGES0, IDX_ROWS0, N_GROUPS0)

    @pl.when(cid == 1)
    def _():
        run(CORE1_ROW0 + sid * (EPT1 // CHUNK), STAGES1, IDX_ROWS1, N_GROUPS1)

    plsc.subcore_barrier()
    pltpu.sync_copy(acc.at[pl.ds(sid * ROWS_PER_TILE, ROWS_PER_TILE)],
                    out_hbm.at[pl.ds(cid * N_PAD + sid * ROWS_PER_TILE,
                                     ROWS_PER_TILE)])


def _make_segsum():
    mesh = plsc.VectorSubcoreMesh(core_axis_name="c", subcore_axis_name="s",
                                  num_cores=NC, num_subcores=NS)
    return pl.kernel(
        _segsum_body,
        out_type=jax.ShapeDtypeStruct((NC * N_PAD, H), jnp.float32),
        mesh=mesh,
        scratch_types=[
            pltpu.VMEM((IDX_ROWS0, CHUNK), jnp.int32),
            pltpu.VMEM((IDX_ROWS0, CHUNK), jnp.int32),
            pltpu.VMEM((CHUNK, H), jnp.float32),
            pltpu.VMEM((CHUNK, H), jnp.float32),
            pltpu.VMEM_SHARED((N_PAD, H), jnp.float32),
            pltpu.SemaphoreType.DMA,
            pltpu.SemaphoreType.DMA,
        ],
    )


def _dot(a, b):
    return lax.dot_general(a, b, (((1,), (0,)), ((), ())),
                           preferred_element_type=jnp.float32,
                           precision=lax.Precision.HIGHEST)


def _layer_body(x_ref, p0_ref, p1_ref, w_ref, o_ref):
    h = x_ref[...] + p0_ref[...] + p1_ref[...]
    o_ref[...] = jnp.maximum(_dot(h, w_ref[...]), 0.0)


def _final_body(h_ref, p0_ref, p1_ref, w2_ref, wo_ref, b_ref,
                out_ref, feat_ref):
    t = h_ref[...] + p0_ref[...] + p1_ref[...]
    h2 = jnp.maximum(_dot(t, w2_ref[...]), 0.0)
    nrm = jnp.sqrt(jnp.sum(h2 * h2, axis=1, keepdims=True))
    feat = h2 / jnp.maximum(nrm, 1e-12)
    feat_ref[...] = feat
    out_ref[...] = _dot(feat, wo_ref[...]) + b_ref[...]


_BLK = 1000
_GRID = N // _BLK


def _row_spec(width):
    return pl.BlockSpec((_BLK, width), lambda i: (i, 0))


def _full_spec(shape):
    return pl.BlockSpec(shape, lambda i: (0, 0))


@jax.jit
def kernel(x, edge_index, W1, W2, W_out, b_out):
    src = edge_index[0]
    dst = edge_index[1]
    pad = E_PAD - E
    src_p = jnp.concatenate([src, jnp.zeros((pad,), jnp.int32)])
    src_p = src_p.reshape(E_PAD // CHUNK, CHUNK)
    trash = N + (jnp.arange(pad, dtype=jnp.int32) % (N_PAD - N))
    dst_p = jnp.concatenate([dst, trash]).reshape(E_PAD // CHUNK, CHUNK)
    zeros = jnp.zeros((ROWS_PER_TILE, H), jnp.float32)

    segsum = _make_segsum()

    layer = pl.pallas_call(
        _layer_body,
        grid=(_GRID,),
        in_specs=[_row_spec(D), _row_spec(H), _row_spec(H), _full_spec((D, H))],
        out_specs=_row_spec(H),
        out_shape=jax.ShapeDtypeStruct((N, H), jnp.float32),
    )

    final = pl.pallas_call(
        _final_body,
        grid=(_GRID,),
        in_specs=[_row_spec(H), _row_spec(H), _row_spec(H),
                  _full_spec((H, H)), _full_spec((H, C)), _full_spec((1, C))],
        out_specs=[_row_spec(C), _row_spec(H)],
        out_shape=[jax.ShapeDtypeStruct((N, C), jnp.float32),
                   jax.ShapeDtypeStruct((N, H), jnp.float32)],
    )

    p = segsum(src_p, dst_p, x, zeros)
    h1 = layer(x, p[:N], p[N_PAD:N_PAD + N], W1)

    p2 = segsum(src_p, dst_p, h1, zeros)
    out, feat = final(h1, p2[:N], p2[N_PAD:N_PAD + N], W2, W_out,
                      b_out.reshape(1, C))
    return (out, feat)


# speedup vs baseline: 1.0227x; 1.0227x over previous
import jax
import jax.numpy as jnp
from jax import lax
from jax.experimental import pallas as pl
from jax.experimental.pallas import tpu as pltpu
from jax.experimental.pallas import tpu_sc as plsc

N = 10000
E = 320000
D = 128
H = 128
C = 64

NC = 2
NS = 16
NW = NC * NS

N_PAD = 10240
E_PAD = 327680
CHUNK = 128
ROWS_PER_TILE = N_PAD // NS

EPT0 = 18432
EPT1 = (E_PAD - NS * EPT0) // NS
STAGES0 = 3
STAGES1 = 1
IDX_ROWS0 = EPT0 // STAGES0 // CHUNK
IDX_ROWS1 = 16
N_GROUPS0 = IDX_ROWS0 // 2
N_GROUPS1 = IDX_ROWS1 // 2
CORE1_ROW0 = NS * EPT0 // CHUNK


def _segsum_body(src_hbm, dst_hbm, table_hbm, zeros_hbm, out_hbm,
                 src_idx, dst_idx, rows0, rows1, acc, sem0, sem1):
    cid = lax.axis_index("c")
    sid = lax.axis_index("s")

    pltpu.sync_copy(zeros_hbm, acc.at[pl.ds(sid * ROWS_PER_TILE, ROWS_PER_TILE)])
    plsc.subcore_barrier()

    def fire(c, slot, sem):
        pltpu.async_copy(table_hbm.at[src_idx.at[c]], slot, sem)

    def drain(c, slot, sem):
        pltpu.make_async_copy(table_hbm.at[src_idx.at[c]], slot, sem).wait()

    def scatter(c, slot):
        pltpu.sync_copy(slot, acc.at[dst_idx.at[c]], add=True)

    def run(base_row, n_stages, idx_rows, n_groups):
        for stage in range(n_stages):
            idx_row0 = base_row + stage * idx_rows
            pltpu.sync_copy(src_hbm.at[pl.ds(idx_row0, idx_rows)],
                            src_idx.at[pl.ds(0, idx_rows)])
            pltpu.sync_copy(dst_hbm.at[pl.ds(idx_row0, idx_rows)],
                            dst_idx.at[pl.ds(0, idx_rows)])

            fire(0, rows0, sem0)

            def group_body(g, carry):
                c0 = 2 * g
                fire(c0 + 1, rows1, sem1)
                drain(c0, rows0, sem0)
                scatter(c0, rows0)

                @pl.when(g < n_groups - 1)
                def _():
                    fire(c0 + 2, rows0, sem0)

                drain(c0 + 1, rows1, sem1)
                scatter(c0 + 1, rows1)
                return carry

            lax.fori_loop(0, n_groups, group_body, 0)

    @pl.when(cid == 0)
    def _():
        run(sid * (EPT0 // CHUNK), STAGES0, IDX_ROWS0, N_GROUPS0)

    @pl.when(cid == 1)
    def _():
        run(CORE1_ROW0 + sid * (EPT1 // CHUNK), STAGES1, IDX_ROWS1, N_GROUPS1)

    plsc.subcore_barrier()
    pltpu.sync_copy(acc.at[pl.ds(sid * ROWS_PER_TILE, ROWS_PER_TILE)],
                    out_hbm.at[pl.ds(cid * N_PAD + sid * ROWS_PER_TILE,
                                     ROWS_PER_TILE)])


def _make_segsum():
    mesh = plsc.VectorSubcoreMesh(core_axis_name="c", subcore_axis_name="s",
                                  num_cores=NC, num_subcores=NS)
    return pl.kernel(
        _segsum_body,
        out_type=jax.ShapeDtypeStruct((NC * N_PAD, H), jnp.float32),
        mesh=mesh,
        scratch_types=[
            pltpu.VMEM((IDX_ROWS0, CHUNK), jnp.int32),
            pltpu.VMEM((IDX_ROWS0, CHUNK), jnp.int32),
            pltpu.VMEM((CHUNK, H), jnp.float32),
            pltpu.VMEM((CHUNK, H), jnp.float32),
            pltpu.VMEM_SHARED((N_PAD, H), jnp.float32),
            pltpu.SemaphoreType.DMA,
            pltpu.SemaphoreType.DMA,
        ],
    )


def _dot(a, b):
    return lax.dot_general(a, b, (((1,), (0,)), ((), ())),
                           preferred_element_type=jnp.float32,
                           precision=lax.Precision.HIGHEST)


def _layer_body(x_ref, p0_ref, p1_ref, w_ref, o_ref):
    h = x_ref[...] + p0_ref[...] + p1_ref[...]
    o_ref[...] = jnp.maximum(_dot(h, w_ref[...]), 0.0)


def _final_body(h_ref, p0_ref, p1_ref, w2_ref, wo_ref, b_ref,
                out_ref, feat_ref):
    t = h_ref[...] + p0_ref[...] + p1_ref[...]
    h2 = jnp.maximum(_dot(t, w2_ref[...]), 0.0)
    nrm = jnp.sqrt(jnp.sum(h2 * h2, axis=1, keepdims=True))
    feat = h2 / jnp.maximum(nrm, 1e-12)
    feat_ref[...] = feat
    out_ref[...] = _dot(feat, wo_ref[...]) + b_ref[...]


_BLK = 2000
_GRID = N // _BLK


def _row_spec(width):
    return pl.BlockSpec((_BLK, width), lambda i: (i, 0))


def _full_spec(shape):
    return pl.BlockSpec(shape, lambda i: (0, 0))


@jax.jit
def kernel(x, edge_index, W1, W2, W_out, b_out):
    src = edge_index[0]
    dst = edge_index[1]
    pad = E_PAD - E
    src_p = jnp.concatenate([src, jnp.zeros((pad,), jnp.int32)])
    src_p = src_p.reshape(E_PAD // CHUNK, CHUNK)
    trash = N + (jnp.arange(pad, dtype=jnp.int32) % (N_PAD - N))
    dst_p = jnp.concatenate([dst, trash]).reshape(E_PAD // CHUNK, CHUNK)
    zeros = jnp.zeros((ROWS_PER_TILE, H), jnp.float32)

    segsum = _make_segsum()

    layer = pl.pallas_call(
        _layer_body,
        grid=(_GRID,),
        in_specs=[_row_spec(D), _row_spec(H), _row_spec(H), _full_spec((D, H))],
        out_specs=_row_spec(H),
        out_shape=jax.ShapeDtypeStruct((N, H), jnp.float32),
    )

    final = pl.pallas_call(
        _final_body,
        grid=(_GRID,),
        in_specs=[_row_spec(H), _row_spec(H), _row_spec(H),
                  _full_spec((H, H)), _full_spec((H, C)), _full_spec((1, C))],
        out_specs=[_row_spec(C), _row_spec(H)],
        out_shape=[jax.ShapeDtypeStruct((N, C), jnp.float32),
                   jax.ShapeDtypeStruct((N, H), jnp.float32)],
    )

    p = segsum(src_p, dst_p, x, zeros)
    h1 = layer(x, p[:N], p[N_PAD:N_PAD + N], W1)

    p2 = segsum(src_p, dst_p, h1, zeros)
    out, feat = final(h1, p2[:N], p2[N_PAD:N_PAD + N], W2, W_out,
                      b_out.reshape(1, C))
    return (out, feat)
